# baseline (speedup 1.0000x reference)
import functools

import jax
import jax.numpy as jnp
from jax import lax
from jax.experimental import pallas as pl
from jax.experimental.pallas import tpu as pltpu
from jax.experimental.pallas import tpu_sc as plsc

N = 10000
D = 128
E = 320000

NC = 2
NS = 16
NW = NC * NS

NP = 10240
CHUNK = 128
CHUNKS_PER_W = 80
EW = CHUNKS_PER_W * CHUNK
EP = NW * EW
ROWS_PER_TILE = NP // NS

CH0 = 120
CH1 = 40

BM = 1024
GRID = NP // BM

_mesh = plsc.VectorSubcoreMesh(core_axis_name="c", subcore_axis_name="s")


@functools.partial(
    pl.kernel,
    out_type=jax.ShapeDtypeStruct((NW, NP), jnp.float32),
    mesh=_mesh,
    compiler_params=pltpu.CompilerParams(needs_layout_passes=False),
    scratch_types=[
        pltpu.VMEM((CHUNKS_PER_W, CHUNK), jnp.int32),
        pltpu.VMEM((NP,), jnp.float32),
    ],
)
def _deg_kernel(dst_hbm, zeros_hbm, out_hbm, idx_v, hist_v):
    c = lax.axis_index("c")
    s = lax.axis_index("s")
    wid = s * NC + c
    pltpu.sync_copy(zeros_hbm, hist_v)
    pltpu.sync_copy(dst_hbm.at[pl.ds(wid * CHUNKS_PER_W, CHUNKS_PER_W)], idx_v)
    ones = jnp.ones((16,), jnp.float32)

    def body(i, carry):
        r = i // (CHUNK // 16)
        k = i % (CHUNK // 16)
        idx = idx_v[r, pl.ds(k * 16, 16)]
        plsc.addupdate_scatter(hist_v, [idx], ones)
        return carry

    lax.fori_loop(0, EW // 16, body, 0, unroll=8)
    pltpu.sync_copy(hist_v, out_hbm.at[wid])


@functools.partial(
    pl.kernel,
    out_type=jax.ShapeDtypeStruct((NC, NP, D), jnp.float32),
    mesh=_mesh,
    compiler_params=pltpu.CompilerParams(needs_layout_passes=False),
    scratch_types=[
        [pltpu.VMEM((CHUNK,), jnp.int32) for _ in range(2)],
        [pltpu.VMEM((CHUNK,), jnp.int32) for _ in range(2)],
        [pltpu.VMEM((CHUNK, D), jnp.float32) for _ in range(2)],
        pltpu.VMEM_SHARED((NP, D), jnp.float32),
        [pltpu.SemaphoreType.DMA for _ in range(2)],
        [pltpu.SemaphoreType.DMA for _ in range(2)],
        [pltpu.SemaphoreType.DMA for _ in range(2)],
        [pltpu.SemaphoreType.DMA for _ in range(2)],
    ],
)
def _agg_kernel(g_hbm, src_hbm, dst_hbm, zrow_hbm, out_hbm,
                sbuf, dbuf, rows, table, gsem, gsem2, isem, dsem):
    c = lax.axis_index("c")
    s = lax.axis_index("s")

    cnt = lax.select(c == 0, CH0, CH1)
    start = s * (CH0 + CH1) + lax.select(c == 0, 0, CH0)

    pltpu.sync_copy(zrow_hbm, table.at[pl.ds(s * ROWS_PER_TILE, ROWS_PER_TILE)])
    plsc.subcore_barrier()

    def start_sidx(q, b):
        pltpu.async_copy(src_hbm.at[q], sbuf[b], isem[b])

    def wait_sidx(b):
        pltpu.make_async_copy(src_hbm.at[0], sbuf[b], isem[b]).wait()

    def start_didx(q, b):
        pltpu.async_copy(dst_hbm.at[q], dbuf[b], dsem[b])

    def wait_didx(b):
        pltpu.make_async_copy(dst_hbm.at[0], dbuf[b], dsem[b]).wait()

    def start_gather(b):
        pltpu.async_copy(g_hbm.at[sbuf[b].at[pl.ds(0, CHUNK // 2)]],
                         rows[b].at[pl.ds(0, CHUNK // 2)], gsem[b])
        pltpu.async_copy(g_hbm.at[sbuf[b].at[pl.ds(CHUNK // 2, CHUNK // 2)]],
                         rows[b].at[pl.ds(CHUNK // 2, CHUNK // 2)], gsem2[b])

    def wait_gather(b):
        pltpu.make_async_copy(g_hbm.at[sbuf[b].at[pl.ds(0, CHUNK // 2)]],
                              rows[b].at[pl.ds(0, CHUNK // 2)], gsem[b]).wait()
        pltpu.make_async_copy(g_hbm.at[sbuf[b].at[pl.ds(CHUNK // 2, CHUNK // 2)]],
                              rows[b].at[pl.ds(CHUNK // 2, CHUNK // 2)],
                              gsem2[b]).wait()

    def substep(q, jr, b):
        nb = 1 - b
        wait_gather(b)

        @pl.when(jr + 1 < cnt)
        def _():
            wait_sidx(nb)
            start_gather(nb)

        wait_didx(b)
        pltpu.sync_copy(rows[b], table.at[dbuf[b]], add=True)

        @pl.when(jr + 2 < cnt)
        def _():
            start_sidx(q + 2, b)
            start_didx(q + 2, b)

    start_sidx(start, 0)
    start_sidx(start + 1, 1)
    start_didx(start, 0)
    start_didx(start + 1, 1)
    wait_sidx(0)
    start_gather(0)

    def grp(g, carry):
        q = start + 2 * g
        substep(q, 2 * g, 0)
        substep(q + 1, 2 * g + 1, 1)
        return carry

    lax.fori_loop(0, cnt // 2, grp, 0)

    plsc.subcore_barrier()
    pltpu.sync_copy(table.at[pl.ds(s * ROWS_PER_TILE, ROWS_PER_TILE)],
                    out_hbm.at[c, pl.ds(s * ROWS_PER_TILE, ROWS_PER_TILE)])


def _tc1_body(degp_ref, x_ref, w_ref, g_ref, dis_ref):
    deg = jnp.sum(degp_ref[...], axis=0) + 1.0
    dis = lax.rsqrt(deg)
    h = jnp.dot(x_ref[...], w_ref[...], preferred_element_type=jnp.float32)
    g_ref[...] = h * dis[:, None]
    dis_ref[...] = dis[:, None]


def _tc1(deg_parts, x_pad, W1):
    return pl.pallas_call(
        _tc1_body,
        grid=(GRID,),
        in_specs=[
            pl.BlockSpec((NW, BM), lambda i: (0, i)),
            pl.BlockSpec((BM, D), lambda i: (i, 0)),
            pl.BlockSpec((D, D), lambda i: (0, 0)),
        ],
        out_specs=[
            pl.BlockSpec((BM, D), lambda i: (i, 0)),
            pl.BlockSpec((BM, 1), lambda i: (i, 0)),
        ],
        out_shape=[
            jax.ShapeDtypeStruct((NP, D), jnp.float32),
            jax.ShapeDtypeStruct((NP, 1), jnp.float32),
        ],
    )(deg_parts, x_pad, W1)


def _tc2_body(p_ref, g1_ref, dis_ref, w_ref, b_ref, g2_ref):
    dis = dis_ref[...]
    agg = p_ref[0] + p_ref[1] + g1_ref[...]
    out1 = jnp.maximum(dis * agg + b_ref[...], 0.0)
    rows = (lax.broadcasted_iota(jnp.int32, (BM, 1), 0)
            + pl.program_id(0) * BM)
    out1 = jnp.where(rows < N, out1, 0.0)
    h2 = jnp.dot(out1, w_ref[...], preferred_element_type=jnp.float32)
    g2_ref[...] = h2 * dis


def _tc2(parts, g1, dis, W2, b1):
    return pl.pallas_call(
        _tc2_body,
        grid=(GRID,),
        in_specs=[
            pl.BlockSpec((NC, BM, D), lambda i: (0, i, 0)),
            pl.BlockSpec((BM, D), lambda i: (i, 0)),
            pl.BlockSpec((BM, 1), lambda i: (i, 0)),
            pl.BlockSpec((D, D), lambda i: (0, 0)),
            pl.BlockSpec((1, D), lambda i: (0, 0)),
        ],
        out_specs=pl.BlockSpec((BM, D), lambda i: (i, 0)),
        out_shape=jax.ShapeDtypeStruct((NP, D), jnp.float32),
    )(parts, g1, dis, W2, b1)


def _tc3_body(q_ref, g2_ref, dis_ref, b_ref, o_ref):
    o_ref[...] = (dis_ref[...] * (q_ref[0] + q_ref[1] + g2_ref[...])
                  + b_ref[...])


def _tc3(parts, g2, dis, b2):
    return pl.pallas_call(
        _tc3_body,
        grid=(GRID,),
        in_specs=[
            pl.BlockSpec((NC, BM, D), lambda i: (0, i, 0)),
            pl.BlockSpec((BM, D), lambda i: (i, 0)),
            pl.BlockSpec((BM, 1), lambda i: (i, 0)),
            pl.BlockSpec((1, D), lambda i: (0, 0)),
        ],
        out_specs=pl.BlockSpec((BM, D), lambda i: (i, 0)),
        out_shape=jax.ShapeDtypeStruct((NP, D), jnp.float32),
    )(parts, g2, dis, b2)


def kernel(x, edge_index, W1, b1, W2, b2):
    src = edge_index[0].astype(jnp.int32)
    dst = edge_index[1].astype(jnp.int32)
    pad = jnp.full((EP - E,), N, dtype=jnp.int32)
    src_p = jnp.concatenate([src, pad]).reshape(NW * CHUNKS_PER_W, CHUNK)
    dst_p = jnp.concatenate([dst, pad]).reshape(NW * CHUNKS_PER_W, CHUNK)
    x_pad = jnp.zeros((NP, D), jnp.float32).at[:N].set(x)
    b1r = b1.reshape(1, D).astype(jnp.float32)
    b2r = b2.reshape(1, D).astype(jnp.float32)
    zeros_np = jnp.zeros((NP,), jnp.float32)
    zrow = jnp.zeros((ROWS_PER_TILE, D), jnp.float32)

    deg_parts = _deg_kernel(dst_p, zeros_np)
    g1, dis = _tc1(deg_parts, x_pad, W1.astype(jnp.float32))
    parts1 = _agg_kernel(g1, src_p, dst_p, zrow)
    g2 = _tc2(parts1, g1, dis, W2.astype(jnp.float32), b1r)
    parts2 = _agg_kernel(g2, src_p, dst_p, zrow)
    out = _tc3(parts2, g2, dis, b2r)
    return out[:N]


# speedup vs baseline: 1.0011x; 1.0011x over previous
import functools

import jax
import jax.numpy as jnp
from jax import lax
from jax.experimental import pallas as pl
from jax.experimental.pallas import tpu as pltpu
from jax.experimental.pallas import tpu_sc as plsc

N = 10000
D = 128
E = 320000

NC = 2
NS = 16
NW = NC * NS

NP = 10240
CHUNK = 128
CHUNKS_PER_W = 80
EW = CHUNKS_PER_W * CHUNK
EP = NW * EW
ROWS_PER_TILE = NP // NS

CH0 = 120
CH1 = 40

BM = 1024
GRID = NP // BM

_mesh = plsc.VectorSubcoreMesh(core_axis_name="c", subcore_axis_name="s")


@functools.partial(
    pl.kernel,
    out_type=jax.ShapeDtypeStruct((NW, NP), jnp.float32),
    mesh=_mesh,
    compiler_params=pltpu.CompilerParams(needs_layout_passes=False),
    scratch_types=[
        pltpu.VMEM((CHUNKS_PER_W, CHUNK), jnp.int32),
        pltpu.VMEM((NP,), jnp.float32),
    ],
)
def _deg_kernel(dst_hbm, zeros_hbm, out_hbm, idx_v, hist_v):
    c = lax.axis_index("c")
    s = lax.axis_index("s")
    wid = s * NC + c
    pltpu.sync_copy(zeros_hbm, hist_v)
    pltpu.sync_copy(dst_hbm.at[pl.ds(wid * CHUNKS_PER_W, CHUNKS_PER_W)], idx_v)
    ones = jnp.ones((16,), jnp.float32)

    def body(i, carry):
        r = i // (CHUNK // 16)
        k = i % (CHUNK // 16)
        idx = idx_v[r, pl.ds(k * 16, 16)]
        plsc.addupdate_scatter(hist_v, [idx], ones)
        return carry

    lax.fori_loop(0, EW // 16, body, 0, unroll=8)
    pltpu.sync_copy(hist_v, out_hbm.at[wid])


@functools.partial(
    pl.kernel,
    out_type=jax.ShapeDtypeStruct((NC, NP, D), jnp.float32),
    mesh=_mesh,
    compiler_params=pltpu.CompilerParams(needs_layout_passes=False),
    scratch_types=[
        [pltpu.VMEM((CHUNK,), jnp.int32) for _ in range(2)],
        [pltpu.VMEM((CHUNK,), jnp.int32) for _ in range(2)],
        [pltpu.VMEM((CHUNK, D), jnp.float32) for _ in range(2)],
        pltpu.VMEM_SHARED((NP, D), jnp.float32),
        [pltpu.SemaphoreType.DMA for _ in range(2)],
        [pltpu.SemaphoreType.DMA for _ in range(2)],
        [pltpu.SemaphoreType.DMA for _ in range(2)],
    ],
)
def _agg_kernel(g_hbm, src_hbm, dst_hbm, zrow_hbm, out_hbm,
                sbuf, dbuf, rows, table, gsem, isem, dsem):
    c = lax.axis_index("c")
    s = lax.axis_index("s")

    cnt = lax.select(c == 0, CH0, CH1)
    start = s * (CH0 + CH1) + lax.select(c == 0, 0, CH0)

    pltpu.sync_copy(zrow_hbm, table.at[pl.ds(s * ROWS_PER_TILE, ROWS_PER_TILE)])
    plsc.subcore_barrier()

    def start_sidx(q, b):
        pltpu.async_copy(src_hbm.at[q], sbuf[b], isem[b])

    def wait_sidx(b):
        pltpu.make_async_copy(src_hbm.at[0], sbuf[b], isem[b]).wait()

    def start_didx(q, b):
        pltpu.async_copy(dst_hbm.at[q], dbuf[b], dsem[b])

    def wait_didx(b):
        pltpu.make_async_copy(dst_hbm.at[0], dbuf[b], dsem[b]).wait()

    def start_gather(b):
        pltpu.async_copy(g_hbm.at[sbuf[b]], rows[b], gsem[b])

    def wait_gather(b):
        pltpu.make_async_copy(g_hbm.at[sbuf[b]], rows[b], gsem[b]).wait()

    def substep(q, jr, b):
        nb = 1 - b
        wait_gather(b)

        @pl.when(jr + 1 < cnt)
        def _():
            wait_sidx(nb)
            start_gather(nb)

        wait_didx(b)
        pltpu.sync_copy(rows[b], table.at[dbuf[b]], add=True)

        @pl.when(jr + 2 < cnt)
        def _():
            start_sidx(q + 2, b)
            start_didx(q + 2, b)

    start_sidx(start, 0)
    start_sidx(start + 1, 1)
    start_didx(start, 0)
    start_didx(start + 1, 1)
    wait_sidx(0)
    start_gather(0)

    def grp(g, carry):
        q = start + 2 * g
        substep(q, 2 * g, 0)
        substep(q + 1, 2 * g + 1, 1)
        return carry

    lax.fori_loop(0, cnt // 2, grp, 0)

    plsc.subcore_barrier()
    pltpu.sync_copy(table.at[pl.ds(s * ROWS_PER_TILE, ROWS_PER_TILE)],
                    out_hbm.at[c, pl.ds(s * ROWS_PER_TILE, ROWS_PER_TILE)])


def _tc1_body(degp_ref, x_ref, w_ref, g_ref, dis_ref):
    deg = jnp.sum(degp_ref[...], axis=0) + 1.0
    dis = lax.rsqrt(deg)
    h = jnp.dot(x_ref[...], w_ref[...], preferred_element_type=jnp.float32)
    g_ref[...] = h * dis[:, None]
    dis_ref[...] = dis[:, None]


def _tc1(deg_parts, x_pad, W1):
    return pl.pallas_call(
        _tc1_body,
        grid=(GRID,),
        in_specs=[
            pl.BlockSpec((NW, BM), lambda i: (0, i)),
            pl.BlockSpec((BM, D), lambda i: (i, 0)),
            pl.BlockSpec((D, D), lambda i: (0, 0)),
        ],
        out_specs=[
            pl.BlockSpec((BM, D), lambda i: (i, 0)),
            pl.BlockSpec((BM, 1), lambda i: (i, 0)),
        ],
        out_shape=[
            jax.ShapeDtypeStruct((NP, D), jnp.float32),
            jax.ShapeDtypeStruct((NP, 1), jnp.float32),
        ],
    )(deg_parts, x_pad, W1)


def _tc2_body(p_ref, g1_ref, dis_ref, w_ref, b_ref, g2_ref):
    dis = dis_ref[...]
    agg = p_ref[0] + p_ref[1] + g1_ref[...]
    out1 = jnp.maximum(dis * agg + b_ref[...], 0.0)
    rows = (lax.broadcasted_iota(jnp.int32, (BM, 1), 0)
            + pl.program_id(0) * BM)
    out1 = jnp.where(rows < N, out1, 0.0)
    h2 = jnp.dot(out1, w_ref[...], preferred_element_type=jnp.float32)
    g2_ref[...] = h2 * dis


def _tc2(parts, g1, dis, W2, b1):
    return pl.pallas_call(
        _tc2_body,
        grid=(GRID,),
        in_specs=[
            pl.BlockSpec((NC, BM, D), lambda i: (0, i, 0)),
            pl.BlockSpec((BM, D), lambda i: (i, 0)),
            pl.BlockSpec((BM, 1), lambda i: (i, 0)),
            pl.BlockSpec((D, D), lambda i: (0, 0)),
            pl.BlockSpec((1, D), lambda i: (0, 0)),
        ],
        out_specs=pl.BlockSpec((BM, D), lambda i: (i, 0)),
        out_shape=jax.ShapeDtypeStruct((NP, D), jnp.float32),
    )(parts, g1, dis, W2, b1)


def _tc3_body(q_ref, g2_ref, dis_ref, b_ref, o_ref):
    o_ref[...] = (dis_ref[...] * (q_ref[0] + q_ref[1] + g2_ref[...])
                  + b_ref[...])


def _tc3(parts, g2, dis, b2):
    return pl.pallas_call(
        _tc3_body,
        grid=(GRID,),
        in_specs=[
            pl.BlockSpec((NC, BM, D), lambda i: (0, i, 0)),
            pl.BlockSpec((BM, D), lambda i: (i, 0)),
            pl.BlockSpec((BM, 1), lambda i: (i, 0)),
            pl.BlockSpec((1, D), lambda i: (0, 0)),
        ],
        out_specs=pl.BlockSpec((BM, D), lambda i: (i, 0)),
        out_shape=jax.ShapeDtypeStruct((NP, D), jnp.float32),
    )(parts, g2, dis, b2)


def kernel(x, edge_index, W1, b1, W2, b2):
    src = edge_index[0].astype(jnp.int32)
    dst = edge_index[1].astype(jnp.int32)
    pad = jnp.full((EP - E,), N, dtype=jnp.int32)
    src_p = jnp.concatenate([src, pad]).reshape(NW * CHUNKS_PER_W, CHUNK)
    dst_p = jnp.concatenate([dst, pad]).reshape(NW * CHUNKS_PER_W, CHUNK)
    x_pad = jnp.zeros((NP, D), jnp.float32).at[:N].set(x)
    b1r = b1.reshape(1, D).astype(jnp.float32)
    b2r = b2.reshape(1, D).astype(jnp.float32)
    zeros_np = jnp.zeros((NP,), jnp.float32)
    zrow = jnp.zeros((ROWS_PER_TILE, D), jnp.float32)

    deg_parts = _deg_kernel(dst_p, zeros_np)
    g1, dis = _tc1(deg_parts, x_pad, W1.astype(jnp.float32))
    parts1 = _agg_kernel(g1, src_p, dst_p, zrow)
    g2 = _tc2(parts1, g1, dis, W2.astype(jnp.float32), b1r)
    parts2 = _agg_kernel(g2, src_p, dst_p, zrow)
    out = _tc3(parts2, g2, dis, b2r)
    return out[:N]


# speedup vs baseline: 1.0330x; 1.0319x over previous
import functools

import jax
import jax.numpy as jnp
from jax import lax
from jax.experimental import pallas as pl
from jax.experimental.pallas import tpu as pltpu
from jax.experimental.pallas import tpu_sc as plsc

N = 10000
D = 128
E = 320000

NC = 2
NS = 16
NW = NC * NS

NP = 10240
CHUNK = 128
CHUNKS_PER_W = 80
EW = CHUNKS_PER_W * CHUNK
EP = NW * EW
ROWS_PER_TILE = NP // NS

CH0 = 120
CH1 = 40

BM = 1024
GRID = NP // BM

_mesh = plsc.VectorSubcoreMesh(core_axis_name="c", subcore_axis_name="s")


@functools.partial(
    pl.kernel,
    out_type=jax.ShapeDtypeStruct((NW, NP), jnp.float32),
    mesh=_mesh,
    compiler_params=pltpu.CompilerParams(needs_layout_passes=False),
    scratch_types=[
        pltpu.VMEM((CHUNKS_PER_W, CHUNK), jnp.int32),
        pltpu.VMEM((NP,), jnp.float32),
    ],
)
def _deg_kernel(dst_hbm, zeros_hbm, out_hbm, idx_v, hist_v):
    c = lax.axis_index("c")
    s = lax.axis_index("s")
    wid = s * NC + c
    pltpu.sync_copy(zeros_hbm, hist_v)
    pltpu.sync_copy(dst_hbm.at[pl.ds(wid * CHUNKS_PER_W, CHUNKS_PER_W)], idx_v)
    ones = jnp.ones((16,), jnp.float32)

    def body(i, carry):
        r = i // (CHUNK // 16)
        k = i % (CHUNK // 16)
        idx = idx_v[r, pl.ds(k * 16, 16)]
        plsc.addupdate_scatter(hist_v, [idx], ones)
        return carry

    lax.fori_loop(0, EW // 16, body, 0, unroll=8)
    pltpu.sync_copy(hist_v, out_hbm.at[wid])


@functools.partial(
    pl.kernel,
    out_type=jax.ShapeDtypeStruct((NC, NP, D), jnp.float32),
    mesh=_mesh,
    compiler_params=pltpu.CompilerParams(needs_layout_passes=False),
    scratch_types=[
        [pltpu.VMEM((CHUNK,), jnp.int32) for _ in range(2)],
        [pltpu.VMEM((CHUNK,), jnp.int32) for _ in range(2)],
        [pltpu.VMEM((CHUNK, D), jnp.float32) for _ in range(2)],
        pltpu.VMEM_SHARED((NP, D), jnp.float32),
        [pltpu.SemaphoreType.DMA for _ in range(2)],
        [pltpu.SemaphoreType.DMA for _ in range(2)],
        [pltpu.SemaphoreType.DMA for _ in range(2)],
    ],
)
def _agg_kernel(g_hbm, src_hbm, dst_hbm, zrow_hbm, out_hbm,
                sbuf, dbuf, rows, table, gsem, isem, dsem):
    c = lax.axis_index("c")
    s = lax.axis_index("s")

    cnt = lax.select(c == 0, CH0, CH1)
    start = s * (CH0 + CH1) + lax.select(c == 0, 0, CH0)

    pltpu.sync_copy(zrow_hbm, table.at[pl.ds(s * ROWS_PER_TILE, ROWS_PER_TILE)])
    plsc.subcore_barrier()

    def start_sidx(q, b):
        pltpu.async_copy(src_hbm.at[q], sbuf[b], isem[b])

    def wait_sidx(b):
        pltpu.make_async_copy(src_hbm.at[0], sbuf[b], isem[b]).wait()

    def start_didx(q, b):
        pltpu.async_copy(dst_hbm.at[q], dbuf[b], dsem[b])

    def wait_didx(b):
        pltpu.make_async_copy(dst_hbm.at[0], dbuf[b], dsem[b]).wait()

    def start_gather(b):
        pltpu.async_copy(g_hbm.at[sbuf[b]], rows[b], gsem[b])

    def wait_gather(b):
        pltpu.make_async_copy(g_hbm.at[sbuf[b]], rows[b], gsem[b]).wait()

    def substep(q, jr, b):
        nb = 1 - b
        wait_gather(b)

        @pl.when(jr + 1 < cnt)
        def _():
            wait_sidx(nb)
            start_gather(nb)

        wait_didx(b)
        pltpu.sync_copy(rows[b], table.at[dbuf[b]], add=True)

        @pl.when(jr + 2 < cnt)
        def _():
            start_sidx(q + 2, b)
            start_didx(q + 2, b)

    start_sidx(start, 0)
    start_sidx(start + 1, 1)
    start_didx(start, 0)
    start_didx(start + 1, 1)
    wait_sidx(0)
    start_gather(0)

    def grp(g, carry):
        q = start + 2 * g
        substep(q, 2 * g, 0)
        substep(q + 1, 2 * g + 1, 1)
        return carry

    lax.fori_loop(0, cnt // 2, grp, 0)

    plsc.subcore_barrier()
    pltpu.sync_copy(table.at[pl.ds(s * ROWS_PER_TILE, ROWS_PER_TILE)],
                    out_hbm.at[c, pl.ds(s * ROWS_PER_TILE, ROWS_PER_TILE)])


def _tc1a_body(x_ref, w_ref, h_ref):
    h_ref[...] = jnp.dot(x_ref[...], w_ref[...],
                         preferred_element_type=jnp.float32)


def _tc1a(x_pad, W1):
    return pl.pallas_call(
        _tc1a_body,
        grid=(GRID,),
        in_specs=[
            pl.BlockSpec((BM, D), lambda i: (i, 0)),
            pl.BlockSpec((D, D), lambda i: (0, 0)),
        ],
        out_specs=pl.BlockSpec((BM, D), lambda i: (i, 0)),
        out_shape=jax.ShapeDtypeStruct((NP, D), jnp.float32),
    )(x_pad, W1)


def _tc1b_body(degp_ref, h_ref, g_ref, dis_ref):
    deg = jnp.sum(degp_ref[...], axis=0) + 1.0
    dis = lax.rsqrt(deg)
    g_ref[...] = h_ref[...] * dis[:, None]
    dis_ref[...] = dis[:, None]


def _tc1b(deg_parts, h1):
    return pl.pallas_call(
        _tc1b_body,
        grid=(GRID,),
        in_specs=[
            pl.BlockSpec((NW, BM), lambda i: (0, i)),
            pl.BlockSpec((BM, D), lambda i: (i, 0)),
        ],
        out_specs=[
            pl.BlockSpec((BM, D), lambda i: (i, 0)),
            pl.BlockSpec((BM, 1), lambda i: (i, 0)),
        ],
        out_shape=[
            jax.ShapeDtypeStruct((NP, D), jnp.float32),
            jax.ShapeDtypeStruct((NP, 1), jnp.float32),
        ],
    )(deg_parts, h1)


def _tc2_body(p_ref, g1_ref, dis_ref, w_ref, b_ref, g2_ref):
    dis = dis_ref[...]
    agg = p_ref[0] + p_ref[1] + g1_ref[...]
    out1 = jnp.maximum(dis * agg + b_ref[...], 0.0)
    rows = (lax.broadcasted_iota(jnp.int32, (BM, 1), 0)
            + pl.program_id(0) * BM)
    out1 = jnp.where(rows < N, out1, 0.0)
    h2 = jnp.dot(out1, w_ref[...], preferred_element_type=jnp.float32)
    g2_ref[...] = h2 * dis


def _tc2(parts, g1, dis, W2, b1):
    return pl.pallas_call(
        _tc2_body,
        grid=(GRID,),
        in_specs=[
            pl.BlockSpec((NC, BM, D), lambda i: (0, i, 0)),
            pl.BlockSpec((BM, D), lambda i: (i, 0)),
            pl.BlockSpec((BM, 1), lambda i: (i, 0)),
            pl.BlockSpec((D, D), lambda i: (0, 0)),
            pl.BlockSpec((1, D), lambda i: (0, 0)),
        ],
        out_specs=pl.BlockSpec((BM, D), lambda i: (i, 0)),
        out_shape=jax.ShapeDtypeStruct((NP, D), jnp.float32),
    )(parts, g1, dis, W2, b1)


def _tc3_body(q_ref, g2_ref, dis_ref, b_ref, o_ref):
    o_ref[...] = (dis_ref[...] * (q_ref[0] + q_ref[1] + g2_ref[...])
                  + b_ref[...])


def _tc3(parts, g2, dis, b2):
    return pl.pallas_call(
        _tc3_body,
        grid=(GRID,),
        in_specs=[
            pl.BlockSpec((NC, BM, D), lambda i: (0, i, 0)),
            pl.BlockSpec((BM, D), lambda i: (i, 0)),
            pl.BlockSpec((BM, 1), lambda i: (i, 0)),
            pl.BlockSpec((1, D), lambda i: (0, 0)),
        ],
        out_specs=pl.BlockSpec((BM, D), lambda i: (i, 0)),
        out_shape=jax.ShapeDtypeStruct((NP, D), jnp.float32),
    )(parts, g2, dis, b2)


def kernel(x, edge_index, W1, b1, W2, b2):
    src = edge_index[0].astype(jnp.int32)
    dst = edge_index[1].astype(jnp.int32)
    pad = jnp.full((EP - E,), N, dtype=jnp.int32)
    src_p = jnp.concatenate([src, pad]).reshape(NW * CHUNKS_PER_W, CHUNK)
    dst_p = jnp.concatenate([dst, pad]).reshape(NW * CHUNKS_PER_W, CHUNK)
    x_pad = jnp.zeros((NP, D), jnp.float32).at[:N].set(x)
    b1r = b1.reshape(1, D).astype(jnp.float32)
    b2r = b2.reshape(1, D).astype(jnp.float32)
    zeros_np = jnp.zeros((NP,), jnp.float32)
    zrow = jnp.zeros((ROWS_PER_TILE, D), jnp.float32)

    deg_parts = _deg_kernel(dst_p, zeros_np)
    h1 = _tc1a(x_pad, W1.astype(jnp.float32))
    g1, dis = _tc1b(deg_parts, h1)
    parts1 = _agg_kernel(g1, src_p, dst_p, zrow)
    g2 = _tc2(parts1, g1, dis, W2.astype(jnp.float32), b1r)
    parts2 = _agg_kernel(g2, src_p, dst_p, zrow)
    out = _tc3(parts2, g2, dis, b2r)
    return out[:N]
